# Initial kernel scaffold; baseline (speedup 1.0000x reference)
import jax
import jax.numpy as jnp
from jax.experimental import pallas as pl


def kernel(x, edge_index, W, b, gamma, beta):
    raise NotImplementedError("write your pallas kernel here")


# speedup vs baseline: 21.5961x; 21.5961x over previous
import functools

import jax
import jax.numpy as jnp
from jax import lax
from jax.experimental import pallas as pl
from jax.experimental.pallas import tpu as pltpu
from jax.experimental.pallas import tpu_sc as plsc

N = 10000
E = 320000
D = 128
BN_EPS = 1e-5

NC = 2
NS = 16
NW = NC * NS
CHUNK = 128
N_CHUNKS = E // CHUNK
BASE_CHUNKS = N_CHUNKS // NW
EXTRA = N_CHUNKS % NW
ROWS_PER_TILE = N // NS
ZCHUNK = 125

_mesh = plsc.VectorSubcoreMesh(core_axis_name="c", subcore_axis_name="s")
_sc_params = pltpu.CompilerParams(use_tc_tiling_on_sc=False)


def _worker_id():
    cid = lax.axis_index("c")
    sid = lax.axis_index("s")
    return sid * NC + cid, cid, sid


@functools.partial(
    pl.kernel,
    out_type=jax.ShapeDtypeStruct((NC, N, 16), jnp.float32),
    mesh=_mesh,
    compiler_params=_sc_params,
    scratch_types=[
        pltpu.VMEM((CHUNK,), jnp.int32),
        pltpu.VMEM((CHUNK, 16), jnp.float32),
        pltpu.VMEM((ROWS_PER_TILE, 16), jnp.float32),
        pltpu.VMEM_SHARED((N, 16), jnp.float32),
    ],
)
def _sc_degree(col2d_hbm, deg_hbm, idx_v, ones_v, zeros_v, acc_sh):
    wid, cid, sid = _worker_id()

    def fill(i, _):
        ones_v[i, :] = jnp.ones((16,), jnp.float32)
        return 0

    lax.fori_loop(0, CHUNK, fill, 0)

    def zfill(i, _):
        zeros_v[i, :] = jnp.zeros((16,), jnp.float32)
        return 0

    lax.fori_loop(0, ROWS_PER_TILE, zfill, 0)
    pltpu.sync_copy(zeros_v, acc_sh.at[pl.ds(sid * ROWS_PER_TILE, ROWS_PER_TILE)])
    plsc.subcore_barrier()

    n_w = BASE_CHUNKS + jnp.where(wid < EXTRA, 1, 0)

    def body(k, _):
        chunk = wid + k * NW
        pltpu.sync_copy(col2d_hbm.at[chunk], idx_v)
        pltpu.sync_copy(ones_v, acc_sh.at[idx_v], add=True)
        return 0

    lax.fori_loop(0, n_w, body, 0)
    plsc.subcore_barrier()
    pltpu.sync_copy(
        acc_sh.at[pl.ds(sid * ROWS_PER_TILE, ROWS_PER_TILE)],
        deg_hbm.at[cid, pl.ds(sid * ROWS_PER_TILE, ROWS_PER_TILE)],
    )


@functools.partial(
    pl.kernel,
    out_type=jax.ShapeDtypeStruct((NC, N, D), jnp.float32),
    mesh=_mesh,
    compiler_params=_sc_params,
    scratch_types=[
        pltpu.VMEM((CHUNK,), jnp.int32),
        pltpu.VMEM((CHUNK,), jnp.int32),
        pltpu.VMEM((CHUNK, D), jnp.float32),
        pltpu.VMEM((ZCHUNK, D), jnp.float32),
        pltpu.VMEM_SHARED((N, D), jnp.float32),
        pltpu.SemaphoreType.DMA,
    ],
)
def _sc_aggregate(g_hbm, row2d_hbm, col2d_hbm, out_hbm,
                  ridx_v, cidx_v, rows_v, zeros_v, acc_sh, sem):
    wid, cid, sid = _worker_id()

    def zfill(i, _):
        for j in range(D // 16):
            zeros_v[i, pl.ds(j * 16, 16)] = jnp.zeros((16,), jnp.float32)
        return 0

    lax.fori_loop(0, ZCHUNK, zfill, 0)

    def zcopy(k, _):
        pltpu.sync_copy(
            zeros_v, acc_sh.at[pl.ds(sid * ROWS_PER_TILE + k * ZCHUNK, ZCHUNK)]
        )
        return 0

    lax.fori_loop(0, ROWS_PER_TILE // ZCHUNK, zcopy, 0)
    plsc.subcore_barrier()

    n_w = BASE_CHUNKS + jnp.where(wid < EXTRA, 1, 0)

    def body(k, _):
        chunk = wid + k * NW
        pltpu.sync_copy(row2d_hbm.at[chunk], ridx_v)
        pltpu.sync_copy(col2d_hbm.at[chunk], cidx_v)
        pltpu.async_copy(g_hbm.at[ridx_v], rows_v, sem).wait()
        pltpu.sync_copy(rows_v, acc_sh.at[cidx_v], add=True)
        return 0

    lax.fori_loop(0, n_w, body, 0)
    plsc.subcore_barrier()
    pltpu.sync_copy(
        acc_sh.at[pl.ds(sid * ROWS_PER_TILE, ROWS_PER_TILE)],
        out_hbm.at[cid, pl.ds(sid * ROWS_PER_TILE, ROWS_PER_TILE)],
    )


ROW_BLOCK = 1000


def _tc_linear_body(deg_ref, x_ref, w_ref, g_ref):
    deg = deg_ref[0, :, 0] + deg_ref[1, :, 0] + 1.0
    dinv = lax.rsqrt(deg)
    h = jnp.dot(x_ref[...], w_ref[...], preferred_element_type=jnp.float32)
    g_ref[...] = h * dinv[:, None]


def _tc_linear(deg, x, W):
    grid = N // ROW_BLOCK
    return pl.pallas_call(
        _tc_linear_body,
        grid=(grid,),
        in_specs=[
            pl.BlockSpec((NC, ROW_BLOCK, 16), lambda i: (0, i, 0)),
            pl.BlockSpec((ROW_BLOCK, D), lambda i: (i, 0)),
            pl.BlockSpec((D, D), lambda i: (0, 0)),
        ],
        out_specs=pl.BlockSpec((ROW_BLOCK, D), lambda i: (i, 0)),
        out_shape=jax.ShapeDtypeStruct((N, D), jnp.float32),
    )(deg, x, W)


def _tc_final_body(part_ref, g_ref, deg_ref, b_ref, gam_ref, bet_ref, o_ref):
    deg = deg_ref[0, :, 0] + deg_ref[1, :, 0] + 1.0
    dinv = lax.rsqrt(deg)
    s = part_ref[0] + part_ref[1] + g_ref[...]
    scale = (1.0 / jnp.sqrt(1.0 + BN_EPS))
    o = (s * dinv[:, None] + b_ref[0]) * (gam_ref[0] * scale) + bet_ref[0]
    o_ref[...] = jnp.maximum(o, 0.0)


def _tc_final(part, g, deg, b, gamma, beta):
    grid = N // ROW_BLOCK
    return pl.pallas_call(
        _tc_final_body,
        grid=(grid,),
        in_specs=[
            pl.BlockSpec((NC, ROW_BLOCK, D), lambda i: (0, i, 0)),
            pl.BlockSpec((ROW_BLOCK, D), lambda i: (i, 0)),
            pl.BlockSpec((NC, ROW_BLOCK, 16), lambda i: (0, i, 0)),
            pl.BlockSpec((1, D), lambda i: (0, 0)),
            pl.BlockSpec((1, D), lambda i: (0, 0)),
            pl.BlockSpec((1, D), lambda i: (0, 0)),
        ],
        out_specs=pl.BlockSpec((ROW_BLOCK, D), lambda i: (i, 0)),
        out_shape=jax.ShapeDtypeStruct((N, D), jnp.float32),
    )(part, g, deg, b, gamma, beta)


def kernel(x, edge_index, W, b, gamma, beta):
    row = edge_index[0].astype(jnp.int32).reshape(N_CHUNKS, CHUNK)
    col = edge_index[1].astype(jnp.int32).reshape(N_CHUNKS, CHUNK)
    deg = _sc_degree(col)
    g = _tc_linear(deg, x, W)
    part = _sc_aggregate(g, row, col)
    return _tc_final(part, g, deg, b.reshape(1, D), gamma.reshape(1, D),
                     beta.reshape(1, D))


# speedup vs baseline: 39.7122x; 1.8389x over previous
import functools

import jax
import jax.numpy as jnp
from jax import lax
from jax.experimental import pallas as pl
from jax.experimental.pallas import tpu as pltpu
from jax.experimental.pallas import tpu_sc as plsc

N = 10000
E = 320000
D = 128
BN_EPS = 1e-5

NC = 2
NS = 16
NW = NC * NS
CHUNK = 100
N_CHUNKS = E // CHUNK
NCH = N_CHUNKS // NW
ROWS_PER_TILE = N // NS

_mesh = plsc.VectorSubcoreMesh(core_axis_name="c", subcore_axis_name="s")
_sc_params = pltpu.CompilerParams(use_tc_tiling_on_sc=False)


def _worker_id():
    cid = lax.axis_index("c")
    sid = lax.axis_index("s")
    return sid * NC + cid, cid, sid


@functools.partial(
    pl.kernel,
    out_type=jax.ShapeDtypeStruct((NC, N, 16), jnp.float32),
    mesh=_mesh,
    compiler_params=_sc_params,
    scratch_types=[
        pltpu.VMEM((NCH, CHUNK), jnp.int32),
        pltpu.VMEM((CHUNK, 16), jnp.float32),
        pltpu.VMEM((ROWS_PER_TILE, 16), jnp.float32),
        pltpu.VMEM_SHARED((N, 16), jnp.float32),
    ],
)
def _sc_degree(col2d_hbm, deg_hbm, cbuf, ones_v, zeros_v, acc_sh):
    wid, cid, sid = _worker_id()
    pltpu.sync_copy(col2d_hbm.at[pl.ds(wid * NCH, NCH)], cbuf)

    def fill(i, _):
        ones_v[i, :] = jnp.ones((16,), jnp.float32)
        return 0

    lax.fori_loop(0, CHUNK, fill, 0)

    def zfill(i, _):
        zeros_v[i, :] = jnp.zeros((16,), jnp.float32)
        return 0

    lax.fori_loop(0, ROWS_PER_TILE, zfill, 0)
    pltpu.sync_copy(zeros_v, acc_sh.at[pl.ds(sid * ROWS_PER_TILE, ROWS_PER_TILE)])
    plsc.subcore_barrier()

    def body(k, _):
        pltpu.sync_copy(ones_v, acc_sh.at[cbuf.at[k]], add=True)
        return 0

    lax.fori_loop(0, NCH, body, 0)
    plsc.subcore_barrier()
    pltpu.sync_copy(
        acc_sh.at[pl.ds(sid * ROWS_PER_TILE, ROWS_PER_TILE)],
        deg_hbm.at[cid, pl.ds(sid * ROWS_PER_TILE, ROWS_PER_TILE)],
    )


@functools.partial(
    pl.kernel,
    out_type=jax.ShapeDtypeStruct((NC, N, D), jnp.float32),
    mesh=_mesh,
    compiler_params=_sc_params,
    scratch_types=[
        pltpu.VMEM((NCH, CHUNK), jnp.int32),
        pltpu.VMEM((NCH, CHUNK), jnp.int32),
        pltpu.VMEM((CHUNK, D), jnp.float32),
        pltpu.VMEM((CHUNK, D), jnp.float32),
        pltpu.VMEM_SHARED((N, D), jnp.float32),
        pltpu.SemaphoreType.DMA,
        pltpu.SemaphoreType.DMA,
    ],
)
def _sc_aggregate(g_hbm, row2d_hbm, col2d_hbm, out_hbm,
                  rbuf, cbuf, rows0, rows1, acc_sh, sem0, sem1):
    wid, cid, sid = _worker_id()
    pltpu.sync_copy(row2d_hbm.at[pl.ds(wid * NCH, NCH)], rbuf)
    pltpu.sync_copy(col2d_hbm.at[pl.ds(wid * NCH, NCH)], cbuf)

    def zfill(i, _):
        for j in range(D // 16):
            rows0[i, pl.ds(j * 16, 16)] = jnp.zeros((16,), jnp.float32)
        return 0

    lax.fori_loop(0, CHUNK, zfill, 0)

    def zcopy(k, _):
        pltpu.sync_copy(
            rows0, acc_sh.at[pl.ds(sid * ROWS_PER_TILE + k * CHUNK, CHUNK)]
        )
        return 0

    lax.fori_loop(0, ROWS_PER_TILE // CHUNK, zcopy, 0)
    pltpu.sync_copy(
        rows0.at[pl.ds(0, ROWS_PER_TILE % CHUNK)],
        acc_sh.at[pl.ds(sid * ROWS_PER_TILE + (ROWS_PER_TILE // CHUNK) * CHUNK,
                        ROWS_PER_TILE % CHUNK)],
    )
    plsc.subcore_barrier()

    pltpu.async_copy(g_hbm.at[rbuf.at[0]], rows0, sem0)

    def pair(i, _):
        k0 = 2 * i
        k1 = k0 + 1
        pltpu.async_copy(g_hbm.at[rbuf.at[k1]], rows1, sem1)
        pltpu.make_async_copy(g_hbm.at[rbuf.at[k0]], rows0, sem0).wait()
        pltpu.sync_copy(rows0, acc_sh.at[cbuf.at[k0]], add=True)

        @pl.when(i < NCH // 2 - 1)
        def _():
            pltpu.async_copy(g_hbm.at[rbuf.at[k0 + 2]], rows0, sem0)

        pltpu.make_async_copy(g_hbm.at[rbuf.at[k1]], rows1, sem1).wait()
        pltpu.sync_copy(rows1, acc_sh.at[cbuf.at[k1]], add=True)
        return 0

    lax.fori_loop(0, NCH // 2, pair, 0)
    plsc.subcore_barrier()
    pltpu.sync_copy(
        acc_sh.at[pl.ds(sid * ROWS_PER_TILE, ROWS_PER_TILE)],
        out_hbm.at[cid, pl.ds(sid * ROWS_PER_TILE, ROWS_PER_TILE)],
    )


ROW_BLOCK = 1000


def _tc_linear_body(deg_ref, x_ref, w_ref, g_ref):
    deg = deg_ref[0, :, 0] + deg_ref[1, :, 0] + 1.0
    dinv = lax.rsqrt(deg)
    h = jnp.dot(x_ref[...], w_ref[...], preferred_element_type=jnp.float32)
    g_ref[...] = h * dinv[:, None]


def _tc_linear(deg, x, W):
    grid = N // ROW_BLOCK
    return pl.pallas_call(
        _tc_linear_body,
        grid=(grid,),
        in_specs=[
            pl.BlockSpec((NC, ROW_BLOCK, 16), lambda i: (0, i, 0)),
            pl.BlockSpec((ROW_BLOCK, D), lambda i: (i, 0)),
            pl.BlockSpec((D, D), lambda i: (0, 0)),
        ],
        out_specs=pl.BlockSpec((ROW_BLOCK, D), lambda i: (i, 0)),
        out_shape=jax.ShapeDtypeStruct((N, D), jnp.float32),
    )(deg, x, W)


def _tc_final_body(part_ref, g_ref, deg_ref, b_ref, gam_ref, bet_ref, o_ref):
    deg = deg_ref[0, :, 0] + deg_ref[1, :, 0] + 1.0
    dinv = lax.rsqrt(deg)
    s = part_ref[0] + part_ref[1] + g_ref[...]
    scale = (1.0 / jnp.sqrt(1.0 + BN_EPS))
    o = (s * dinv[:, None] + b_ref[0]) * (gam_ref[0] * scale) + bet_ref[0]
    o_ref[...] = jnp.maximum(o, 0.0)


def _tc_final(part, g, deg, b, gamma, beta):
    grid = N // ROW_BLOCK
    return pl.pallas_call(
        _tc_final_body,
        grid=(grid,),
        in_specs=[
            pl.BlockSpec((NC, ROW_BLOCK, D), lambda i: (0, i, 0)),
            pl.BlockSpec((ROW_BLOCK, D), lambda i: (i, 0)),
            pl.BlockSpec((NC, ROW_BLOCK, 16), lambda i: (0, i, 0)),
            pl.BlockSpec((1, D), lambda i: (0, 0)),
            pl.BlockSpec((1, D), lambda i: (0, 0)),
            pl.BlockSpec((1, D), lambda i: (0, 0)),
        ],
        out_specs=pl.BlockSpec((ROW_BLOCK, D), lambda i: (i, 0)),
        out_shape=jax.ShapeDtypeStruct((N, D), jnp.float32),
    )(part, g, deg, b, gamma, beta)


def kernel(x, edge_index, W, b, gamma, beta):
    row = edge_index[0].astype(jnp.int32).reshape(N_CHUNKS, CHUNK)
    col = edge_index[1].astype(jnp.int32).reshape(N_CHUNKS, CHUNK)
    assert N_CHUNKS == NW * NCH
    deg = _sc_degree(col)
    g = _tc_linear(deg, x, W)
    part = _sc_aggregate(g, row, col)
    return _tc_final(part, g, deg, b.reshape(1, D), gamma.reshape(1, D),
                     beta.reshape(1, D))
